# baseline (speedup 1.0000x reference)
import functools

import jax
import jax.numpy as jnp
from jax import lax
from jax.experimental import pallas as pl
from jax.experimental.pallas import tpu as pltpu
from jax.experimental.pallas import tpu_sc as plsc

B, S, D, E, F, V = 1, 2048, 768, 8, 3072, 32128
T = B * S
TB = 128
P = T + E * TB
W = P // TB
NC, NS = 2, 16
NW = NC * NS
TPW = T // NW
PPW = P // NW

_PREC = jax.lax.Precision.DEFAULT


def _emb_gather_body(tok_hbm, emb_hbm, x_hbm, idx_v, rows_v, sem):
    wid = lax.axis_index("s") * NC + lax.axis_index("c")
    base = wid * TPW
    pltpu.sync_copy(tok_hbm.at[pl.ds(base, TPW)], idx_v)
    pltpu.async_copy(emb_hbm.at[idx_v], rows_v, sem).wait()
    pltpu.sync_copy(rows_v, x_hbm.at[pl.ds(base, TPW)])


def _router_body(x_ref, scale_ref, wr_ref, m_ref, h_ref, pos_ref, eid_ref):
    x = x_ref[...]
    var = jnp.mean(x * x, axis=1, keepdims=True)
    h = x * jax.lax.rsqrt(var + 1e-6) * scale_ref[...]
    logits = jnp.dot(h, wr_ref[...], preferred_element_type=jnp.float32,
                     precision=_PREC)
    mx = jnp.max(logits, axis=1, keepdims=True)
    ex = jnp.exp(logits - mx)
    probs = ex / jnp.sum(ex, axis=1, keepdims=True)
    pmax = jnp.max(probs, axis=1, keepdims=True)
    iot = lax.broadcasted_iota(jnp.int32, (T, E), 1)
    eidx = jnp.min(jnp.where(probs == pmax, iot, E), axis=1, keepdims=True)
    dh = (iot == eidx).astype(jnp.float32)

    counts = jnp.sum(dh, axis=0, keepdims=True)
    pc = (((counts.astype(jnp.int32) + (TB - 1)) // TB) * TB).astype(jnp.float32)
    parts = [jnp.zeros((1, 1), jnp.float32)]
    run = jnp.zeros((1, 1), jnp.float32)
    for e in range(E - 1):
        run = run + pc[:, e:e + 1]
        parts.append(run)
    aoff = jnp.concatenate(parts, axis=1)
    ends = aoff + pc

    ci = lax.broadcasted_iota(jnp.int32, (TB, TB), 0)
    cj = lax.broadcasted_iota(jnp.int32, (TB, TB), 1)
    lc = (cj < ci).astype(jnp.float32)
    base = jnp.zeros((1, E), jnp.float32)
    for i in range(T // TB):
        dhc = dh[i * TB:(i + 1) * TB]
        rank = jnp.dot(lc, dhc, preferred_element_type=jnp.float32,
                       precision=_PREC) + base
        posc = jnp.sum(dhc * (aoff + rank), axis=1, keepdims=True)
        pos_ref[i * TB:(i + 1) * TB, :] = posc.astype(jnp.int32)
        base = base + jnp.sum(dhc, axis=0, keepdims=True)

    gate = pmax * m_ref[...]
    h_ref[...] = h * gate

    ws = lax.broadcasted_iota(jnp.int32, (32, E), 0) * TB
    eidm = jnp.sum((ends.astype(jnp.int32) <= ws).astype(jnp.int32),
                   axis=1, keepdims=True)
    eid_ref[...] = jnp.minimum(eidm, E - 1)


def _router_call(x, scale, wr, mf):
    return pl.pallas_call(
        _router_body,
        out_shape=(
            jax.ShapeDtypeStruct((T, D), jnp.float32),
            jax.ShapeDtypeStruct((T, 1), jnp.int32),
            jax.ShapeDtypeStruct((32, 1), jnp.int32),
        ),
    )(x, scale, wr, mf)


def _sort_gather_body(pos_hbm, hpre_hbm, hs_hbm, pos_v, rows_v, sem):
    wid = lax.axis_index("s") * NC + lax.axis_index("c")
    base = wid * TPW
    pltpu.sync_copy(pos_hbm.at[pl.ds(base, TPW)], pos_v)
    pltpu.sync_copy(hpre_hbm.at[pl.ds(base, TPW)], rows_v)
    pltpu.async_copy(rows_v, hs_hbm.at[pos_v], sem).wait()


NF = 4
FB = F // NF


def _ffn_body(eid_s, h_ref, wi_ref, wo_ref, o_ref):
    f = pl.program_id(1)
    a = jnp.dot(h_ref[...], wi_ref[0], preferred_element_type=jnp.float32,
                precision=_PREC)
    a = jnp.maximum(a, 0.0)
    y = jnp.dot(a, wo_ref[0], preferred_element_type=jnp.float32,
                precision=_PREC)

    @pl.when(f == 0)
    def _():
        o_ref[...] = y

    @pl.when(f != 0)
    def _():
        o_ref[...] += y


def _ffn_call(eid, hs, wi, wo):
    grid_spec = pltpu.PrefetchScalarGridSpec(
        num_scalar_prefetch=1,
        grid=(W, NF),
        in_specs=[
            pl.BlockSpec((TB, D), lambda w, f, eid: (w, 0)),
            pl.BlockSpec((1, D, FB), lambda w, f, eid: (eid[w], 0, f)),
            pl.BlockSpec((1, FB, D), lambda w, f, eid: (eid[w], f, 0)),
        ],
        out_specs=pl.BlockSpec((TB, D), lambda w, f, eid: (w, 0)),
    )
    return pl.pallas_call(
        _ffn_body,
        grid_spec=grid_spec,
        out_shape=jax.ShapeDtypeStruct((P, D), jnp.float32),
        compiler_params=pltpu.CompilerParams(vmem_limit_bytes=120 * 1024 * 1024),
    )(eid, hs, wi, wo)


def _combine_body(pos_hbm, y_hbm, x_hbm, out_hbm, pos_v, y_v, x_v, sem):
    wid = lax.axis_index("s") * NC + lax.axis_index("c")
    base = wid * TPW
    pltpu.sync_copy(pos_hbm.at[pl.ds(base, TPW)], pos_v)
    pltpu.async_copy(y_hbm.at[pos_v], y_v, sem).wait()
    pltpu.sync_copy(x_hbm.at[pl.ds(base, TPW)], x_v)

    def rloop(r, carry):
        for c in range(D // 16):
            x_v[r, pl.ds(c * 16, 16)] = (x_v[r, pl.ds(c * 16, 16)]
                                         + y_v[r, pl.ds(c * 16, 16)])
        return carry
    lax.fori_loop(0, TPW, rloop, 0)
    pltpu.sync_copy(x_v, out_hbm.at[pl.ds(base, TPW)])


@functools.lru_cache(maxsize=None)
def _sc_kernels():
    mesh = plsc.VectorSubcoreMesh(core_axis_name="c", subcore_axis_name="s",
                                  num_cores=NC, num_subcores=NS)
    emb_gather = pl.kernel(
        _emb_gather_body,
        out_type=jax.ShapeDtypeStruct((T, D), jnp.float32),
        mesh=mesh,
        scratch_types=[
            pltpu.VMEM((TPW,), jnp.int32),
            pltpu.VMEM((TPW, D), jnp.float32),
            pltpu.SemaphoreType.DMA,
        ],
    )
    sort_gather = pl.kernel(
        _sort_gather_body,
        out_type=jax.ShapeDtypeStruct((P, D), jnp.float32),
        mesh=mesh,
        scratch_types=[
            pltpu.VMEM((TPW,), jnp.int32),
            pltpu.VMEM((TPW, D), jnp.float32),
            pltpu.SemaphoreType.DMA,
        ],
    )
    combine = pl.kernel(
        _combine_body,
        out_type=jax.ShapeDtypeStruct((T, D), jnp.float32),
        mesh=mesh,
        scratch_types=[
            pltpu.VMEM((TPW,), jnp.int32),
            pltpu.VMEM((TPW, D), jnp.float32),
            pltpu.VMEM((TPW, D), jnp.float32),
            pltpu.SemaphoreType.DMA,
        ],
    )
    return emb_gather, sort_gather, combine


def kernel(input_ids, attention_mask, labels, emb, ln_scale, Wr, wi, wo):
    tok = input_ids.reshape(-1)
    mf = attention_mask.reshape(-1, 1).astype(jnp.float32)
    emb_gather, sort_gather, combine = _sc_kernels()
    x = emb_gather(tok, emb)
    h_pre, pos2, eid2 = _router_call(x, ln_scale.reshape(1, -1), Wr, mf)
    pos = pos2.reshape(-1)
    eid = eid2.reshape(-1)
    hs = sort_gather(pos, h_pre)
    ys = _ffn_call(eid, hs, wi, wo)
    out = combine(pos, ys, x)
    return out.reshape(B, S, D)


# speedup vs baseline: 1.5606x; 1.5606x over previous
import functools

import jax
import jax.numpy as jnp
from jax import lax
from jax.experimental import pallas as pl
from jax.experimental.pallas import tpu as pltpu
from jax.experimental.pallas import tpu_sc as plsc

B, S, D, E, F, V = 1, 2048, 768, 8, 3072, 32128
T = B * S
TB = 128
P = T + E * TB
W = P // TB
NC, NS = 2, 16
NW = NC * NS
TPW = T // NW
PPW = P // NW

_PREC = jax.lax.Precision.DEFAULT


def _emb_gather_body(tok_hbm, emb_hbm, x_hbm, idx_v, rows_v, sem):
    wid = lax.axis_index("s") * NC + lax.axis_index("c")
    base = wid * TPW
    pltpu.sync_copy(tok_hbm.at[pl.ds(base, TPW)], idx_v)
    pltpu.async_copy(emb_hbm.at[idx_v], rows_v, sem).wait()
    pltpu.sync_copy(rows_v, x_hbm.at[pl.ds(base, TPW)])


def _router_body(x_ref, scale_ref, wr_ref, m_ref, h_ref, pos_ref, eid_ref):
    x = x_ref[...]
    var = jnp.mean(x * x, axis=1, keepdims=True)
    h = x * jax.lax.rsqrt(var + 1e-6) * scale_ref[...]
    logits = jnp.dot(h, wr_ref[...], preferred_element_type=jnp.float32,
                     precision=_PREC)
    mx = jnp.max(logits, axis=1, keepdims=True)
    ex = jnp.exp(logits - mx)
    probs = ex / jnp.sum(ex, axis=1, keepdims=True)
    pmax = jnp.max(probs, axis=1, keepdims=True)
    iot = lax.broadcasted_iota(jnp.int32, (T, E), 1)
    eidx = jnp.min(jnp.where(probs == pmax, iot, E), axis=1, keepdims=True)
    dh = (iot == eidx).astype(jnp.float32)

    counts = jnp.sum(dh, axis=0, keepdims=True)
    pc = (((counts.astype(jnp.int32) + (TB - 1)) // TB) * TB).astype(jnp.float32)
    parts = [jnp.zeros((1, 1), jnp.float32)]
    run = jnp.zeros((1, 1), jnp.float32)
    for e in range(E - 1):
        run = run + pc[:, e:e + 1]
        parts.append(run)
    aoff = jnp.concatenate(parts, axis=1)
    ends = aoff + pc

    ci = lax.broadcasted_iota(jnp.int32, (TB, TB), 0)
    cj = lax.broadcasted_iota(jnp.int32, (TB, TB), 1)
    lc = (cj < ci).astype(jnp.float32)
    base = jnp.zeros((1, E), jnp.float32)
    for i in range(T // TB):
        dhc = dh[i * TB:(i + 1) * TB]
        rank = jnp.dot(lc, dhc, preferred_element_type=jnp.float32,
                       precision=_PREC) + base
        posc = jnp.sum(dhc * (aoff + rank), axis=1, keepdims=True)
        pos_ref[i * TB:(i + 1) * TB, :] = posc.astype(jnp.int32)
        base = base + jnp.sum(dhc, axis=0, keepdims=True)

    gate = pmax * m_ref[...]
    h_ref[...] = h * gate

    ws = lax.broadcasted_iota(jnp.int32, (32, E), 0) * TB
    eidm = jnp.sum((ends.astype(jnp.int32) <= ws).astype(jnp.int32),
                   axis=1, keepdims=True)
    eid_col = jnp.minimum(eidm, E - 1)
    used = (ends[:, E - 1:].astype(jnp.int32) // TB) * jnp.ones((32, 1), jnp.int32)
    row = lax.broadcasted_iota(jnp.int32, (32, 1), 0)
    eid_ref[...] = jnp.where(row == 31, used, eid_col)


def _router_call(x, scale, wr, mf):
    return pl.pallas_call(
        _router_body,
        out_shape=(
            jax.ShapeDtypeStruct((T, D), jnp.float32),
            jax.ShapeDtypeStruct((T, 1), jnp.int32),
            jax.ShapeDtypeStruct((32, 1), jnp.int32),
        ),
    )(x, scale, wr, mf)


def _sort_gather_body(pos_hbm, hpre_hbm, hs_hbm, pos_v, rows_v, sem):
    wid = lax.axis_index("s") * NC + lax.axis_index("c")
    base = wid * TPW
    pltpu.sync_copy(pos_hbm.at[pl.ds(base, TPW)], pos_v)
    pltpu.sync_copy(hpre_hbm.at[pl.ds(base, TPW)], rows_v)
    pltpu.async_copy(rows_v, hs_hbm.at[pos_v], sem).wait()


def _ffn_body(eid_s, h_ref, wi_ref, wo_ref, o_ref):
    @pl.when(pl.program_id(0) < eid_s[31])
    def _():
        a = jnp.dot(h_ref[...], wi_ref[0], preferred_element_type=jnp.float32,
                    precision=_PREC)
        a = jnp.maximum(a, 0.0)
        o_ref[...] = jnp.dot(a, wo_ref[0], preferred_element_type=jnp.float32,
                             precision=_PREC)


def _ffn_call(eid, hs, wi, wo):
    grid_spec = pltpu.PrefetchScalarGridSpec(
        num_scalar_prefetch=1,
        grid=(W,),
        in_specs=[
            pl.BlockSpec((TB, D), lambda w, eid: (w, 0)),
            pl.BlockSpec((1, D, F), lambda w, eid: (eid[w], 0, 0)),
            pl.BlockSpec((1, F, D), lambda w, eid: (eid[w], 0, 0)),
        ],
        out_specs=pl.BlockSpec((TB, D), lambda w, eid: (w, 0)),
    )
    return pl.pallas_call(
        _ffn_body,
        grid_spec=grid_spec,
        out_shape=jax.ShapeDtypeStruct((P, D), jnp.float32),
        compiler_params=pltpu.CompilerParams(vmem_limit_bytes=120 * 1024 * 1024),
    )(eid, hs, wi, wo)


def _combine_body(pos_hbm, y_hbm, x_hbm, out_hbm, pos_v, y_v, x_v, sem):
    wid = lax.axis_index("s") * NC + lax.axis_index("c")
    base = wid * TPW
    pltpu.sync_copy(pos_hbm.at[pl.ds(base, TPW)], pos_v)
    pltpu.async_copy(y_hbm.at[pos_v], y_v, sem).wait()
    pltpu.sync_copy(x_hbm.at[pl.ds(base, TPW)], x_v)

    def rloop(r):
        for c in range(D // 16):
            x_v[r, pl.ds(c * 16, 16)] = (x_v[r, pl.ds(c * 16, 16)]
                                         + y_v[r, pl.ds(c * 16, 16)])
    plsc.parallel_loop(0, TPW, 1, unroll=2)(rloop)
    pltpu.sync_copy(x_v, out_hbm.at[pl.ds(base, TPW)])


@functools.lru_cache(maxsize=None)
def _sc_kernels():
    mesh = plsc.VectorSubcoreMesh(core_axis_name="c", subcore_axis_name="s",
                                  num_cores=NC, num_subcores=NS)
    emb_gather = pl.kernel(
        _emb_gather_body,
        out_type=jax.ShapeDtypeStruct((T, D), jnp.float32),
        mesh=mesh,
        scratch_types=[
            pltpu.VMEM((TPW,), jnp.int32),
            pltpu.VMEM((TPW, D), jnp.float32),
            pltpu.SemaphoreType.DMA,
        ],
    )
    sort_gather = pl.kernel(
        _sort_gather_body,
        out_type=jax.ShapeDtypeStruct((P, D), jnp.float32),
        mesh=mesh,
        scratch_types=[
            pltpu.VMEM((TPW,), jnp.int32),
            pltpu.VMEM((TPW, D), jnp.float32),
            pltpu.SemaphoreType.DMA,
        ],
    )
    combine = pl.kernel(
        _combine_body,
        out_type=jax.ShapeDtypeStruct((T, D), jnp.float32),
        mesh=mesh,
        scratch_types=[
            pltpu.VMEM((TPW,), jnp.int32),
            pltpu.VMEM((TPW, D), jnp.float32),
            pltpu.VMEM((TPW, D), jnp.float32),
            pltpu.SemaphoreType.DMA,
        ],
    )
    return emb_gather, sort_gather, combine


def kernel(input_ids, attention_mask, labels, emb, ln_scale, Wr, wi, wo):
    tok = input_ids.reshape(-1)
    mf = attention_mask.reshape(-1, 1).astype(jnp.float32)
    emb_gather, sort_gather, combine = _sc_kernels()
    x = emb_gather(tok, emb)
    h_pre, pos2, eid2 = _router_call(x, ln_scale.reshape(1, -1), Wr, mf)
    pos = pos2.reshape(-1)
    eid = eid2.reshape(-1)
    hs = sort_gather(pos, h_pre)
    ys = _ffn_call(eid, hs, wi, wo)
    out = combine(pos, ys, x)
    return out.reshape(B, S, D)
